# baseline (speedup 1.0000x reference)
import functools

import jax
import jax.numpy as jnp
from jax import lax
from jax.experimental import pallas as pl
from jax.experimental.pallas import tpu as pltpu
from jax.experimental.pallas import tpu_sc as plsc

_EMBED = 128
_CHUNK = 256
_NBUF = 2
_YLOAD = 64


def kernel(author_idx, publisher_idx, year_idx, W_author, W_year, W_publisher):
    batch = author_idx.shape[0]
    n_year = W_year.shape[0]
    info = plsc.get_sparse_core_info()
    num_cores = info.num_cores
    nw = num_cores * info.num_subcores
    b_per_w = batch // nw
    n_chunks = b_per_w // _CHUNK

    mesh = plsc.VectorSubcoreMesh(core_axis_name="c", subcore_axis_name="s")

    @functools.partial(
        pl.kernel,
        out_type=jax.ShapeDtypeStruct((batch, 3 * _EMBED), jnp.float32),
        mesh=mesh,
        scratch_types=[
            pltpu.VMEM((3 * b_per_w,), jnp.int32),
            pltpu.VMEM((_NBUF, _CHUNK, _EMBED), jnp.float32),
            pltpu.VMEM_SHARED((n_year, _EMBED), jnp.float32),
            pltpu.SemaphoreType.DMA,
        ]
        + [pltpu.SemaphoreType.DMA] * (2 * _NBUF),
    )
    def _gather3(a_idx, p_idx, y_idx, wa, wy, wp, out, idx_v, rows_v, yspm, isem, *sems):
        gsems = sems[:_NBUF]
        wsems = sems[_NBUF:]
        sid = lax.axis_index("s")
        wid = sid * num_cores + lax.axis_index("c")
        base = wid * b_per_w

        idx_copies = [
            pltpu.async_copy(
                src.at[pl.ds(base, b_per_w)],
                idx_v.at[pl.ds(r * b_per_w, b_per_w)],
                isem,
            )
            for r, src in enumerate((a_idx, p_idx, y_idx))
        ]

        yoff = jnp.minimum(sid * _YLOAD, n_year - _YLOAD)
        pltpu.sync_copy(
            wy.at[pl.ds(yoff, _YLOAD)], yspm.at[pl.ds(yoff, _YLOAD)]
        )

        for c in idx_copies:
            c.wait()

        tables = (wa, wp, yspm)
        cols = (0, 2 * _EMBED, _EMBED)
        tasks = [(r, j, cols[r]) for r in range(3) for j in range(n_chunks)]
        T = len(tasks)
        first_year = 2 * n_chunks

        def start_gather(t):
            r, j, _ = tasks[t]
            b = t % _NBUF
            return pltpu.async_copy(
                tables[r].at[idx_v.at[pl.ds(r * b_per_w + j * _CHUNK, _CHUNK)]],
                rows_v.at[b],
                gsems[b],
            )

        def start_write(t):
            r, j, col = tasks[t]
            b = t % _NBUF
            return pltpu.async_copy(
                rows_v.at[b],
                out.at[pl.ds(base + j * _CHUNK, _CHUNK), pl.ds(col, _EMBED)],
                wsems[b],
            )

        gcp, wcp = {}, {}
        for t in range(min(_NBUF - 1, T)):
            if t == first_year:
                plsc.subcore_barrier()
            gcp[t] = start_gather(t)
        for t in range(T):
            gcp[t].wait()
            wcp[t] = start_write(t)
            u = t + _NBUF - 1
            if u < T:
                if u - _NBUF >= 0:
                    wcp[u - _NBUF].wait()
                if u == first_year:
                    plsc.subcore_barrier()
                gcp[u] = start_gather(u)
        for t in range(max(0, T - _NBUF), T):
            wcp[t].wait()

    return _gather3(author_idx, publisher_idx, year_idx, W_author, W_year, W_publisher)


# speedup vs baseline: 1.0599x; 1.0599x over previous
import functools

import jax
import jax.numpy as jnp
from jax import lax
from jax.experimental import pallas as pl
from jax.experimental.pallas import tpu as pltpu
from jax.experimental.pallas import tpu_sc as plsc

_EMBED = 128
_CHUNK = 256
_NBUF = 3
_YLOAD = 64


def kernel(author_idx, publisher_idx, year_idx, W_author, W_year, W_publisher):
    batch = author_idx.shape[0]
    n_year = W_year.shape[0]
    info = plsc.get_sparse_core_info()
    num_cores = info.num_cores
    nw = num_cores * info.num_subcores
    b_per_w = batch // nw
    n_chunks = b_per_w // _CHUNK

    mesh = plsc.VectorSubcoreMesh(core_axis_name="c", subcore_axis_name="s")

    @functools.partial(
        pl.kernel,
        out_type=jax.ShapeDtypeStruct((batch, 3 * _EMBED), jnp.float32),
        mesh=mesh,
        scratch_types=[
            pltpu.VMEM((3 * b_per_w,), jnp.int32),
            pltpu.VMEM((_NBUF, _CHUNK, _EMBED), jnp.float32),
            pltpu.VMEM_SHARED((n_year, _EMBED), jnp.float32),
        ]
        + [pltpu.SemaphoreType.DMA] * (2 * _NBUF + 3),
    )
    def _gather3(a_idx, p_idx, y_idx, wa, wy, wp, out, idx_v, rows_v, yspm, *sems):
        gsems = sems[:_NBUF]
        wsems = sems[_NBUF : 2 * _NBUF]
        isems = sems[2 * _NBUF :]
        sid = lax.axis_index("s")
        wid = sid * num_cores + lax.axis_index("c")
        base = wid * b_per_w

        idx_copies = [
            pltpu.async_copy(
                src.at[pl.ds(base, b_per_w)],
                idx_v.at[pl.ds(r * b_per_w, b_per_w)],
                isems[r],
            )
            for r, src in enumerate((a_idx, p_idx, y_idx))
        ]

        yoff = jnp.minimum(sid * _YLOAD, n_year - _YLOAD)
        pltpu.sync_copy(
            wy.at[pl.ds(yoff, _YLOAD)], yspm.at[pl.ds(yoff, _YLOAD)]
        )

        tables = (wa, wp, yspm)
        cols = (0, 2 * _EMBED, _EMBED)
        tasks = [(r, j, cols[r]) for r in range(3) for j in range(n_chunks)]
        T = len(tasks)
        first_year = 2 * n_chunks

        def start_gather(t):
            r, j, _ = tasks[t]
            b = t % _NBUF
            return pltpu.async_copy(
                tables[r].at[idx_v.at[pl.ds(r * b_per_w + j * _CHUNK, _CHUNK)]],
                rows_v.at[b],
                gsems[b],
            )

        def start_write(t):
            r, j, col = tasks[t]
            b = t % _NBUF
            return pltpu.async_copy(
                rows_v.at[b],
                out.at[pl.ds(base + j * _CHUNK, _CHUNK), pl.ds(col, _EMBED)],
                wsems[b],
            )

        def pre_gather(u):
            if u % n_chunks == 0:
                idx_copies[u // n_chunks].wait()
            if u == first_year:
                plsc.subcore_barrier()

        gcp, wcp = {}, {}
        for t in range(min(_NBUF - 1, T)):
            pre_gather(t)
            gcp[t] = start_gather(t)
        for t in range(T):
            gcp[t].wait()
            wcp[t] = start_write(t)
            u = t + _NBUF - 1
            if u < T:
                if u - _NBUF >= 0:
                    wcp[u - _NBUF].wait()
                pre_gather(u)
                gcp[u] = start_gather(u)
        for t in range(max(0, T - _NBUF), T):
            wcp[t].wait()

    return _gather3(author_idx, publisher_idx, year_idx, W_author, W_year, W_publisher)
